# baseline (speedup 1.0000x reference)
import jax
import jax.numpy as jnp
from jax import lax
from jax.experimental import pallas as pl

_TOPK = 8
_BB = 8


def _loss_body(feat_ref, logit_row_ref, logit_col_ref, out_ref):
    a = feat_ref[...]
    nsq = jnp.sum(a * a, axis=1, keepdims=True)
    inv = lax.rsqrt(jnp.maximum(nsq, 1e-24))
    b = a * inv
    s = lax.dot_general(b, b, (((1,), (1,)), ((0,), (0,))),
                        preferred_element_type=jnp.float32)

    p = jax.nn.sigmoid(logit_row_ref[...])
    anchor = jax.nn.sigmoid(logit_col_ref[...])

    m = jnp.max(s, axis=2, keepdims=True)
    for _ in range(_TOPK - 1):
        m = jnp.max(jnp.where(s < m, s, -jnp.inf), axis=2, keepdims=True)
    acc = jnp.sum(jnp.where(s >= m, jnp.abs(anchor - p), 0.0))

    @pl.when(pl.program_id(0) == 0)
    def _init():
        out_ref[...] = jnp.zeros_like(out_ref)

    out_ref[...] += acc


def kernel(feats, logits):
    bsz, c, h, w = feats.shape
    n = h * w
    feat = feats.reshape(bsz, c, n)
    logit_row = logits.reshape(bsz, 1, n)
    logit_col = logits.reshape(bsz, n, 1)
    partial = pl.pallas_call(
        _loss_body,
        grid=(bsz // _BB,),
        in_specs=[
            pl.BlockSpec((_BB, c, n), lambda i: (i, 0, 0)),
            pl.BlockSpec((_BB, 1, n), lambda i: (i, 0, 0)),
            pl.BlockSpec((_BB, n, 1), lambda i: (i, 0, 0)),
        ],
        out_specs=pl.BlockSpec((1, 1, 128), lambda i: (0, 0, 0)),
        out_shape=jax.ShapeDtypeStruct((1, 1, 128), jnp.float32),
    )(feat, logit_row, logit_col)
    return partial[0, 0, 0] / (bsz * n * _TOPK)
